# Initial kernel scaffold; baseline (speedup 1.0000x reference)
import jax
import jax.numpy as jnp
from jax.experimental import pallas as pl


def kernel(x, edge_index, pre_W, pre_b, post_W, post_b, lin_W, lin_b):
    raise NotImplementedError("write your pallas kernel here")


# speedup vs baseline: 3.3571x; 3.3571x over previous
import functools

import jax
import jax.numpy as jnp
import numpy as np
from jax import lax
from jax.experimental import pallas as pl
from jax.experimental.pallas import tpu as pltpu
from jax.experimental.pallas import tpu_sc as plsc

N = 10000
E = 160000
F = 256
L = 3
AVG_DEG = 16.0

_hist = np.array([1.0] * 10 + [float(int(AVG_DEG) * 2)] * 10)
_bins = np.arange(_hist.shape[0])
ADL = float((np.log(_bins + 1.0) * _hist).sum() / _hist.sum())

NP = 10240
G = 32
NGRP = NP // G
NW = 32
GPW = NGRP // NW
CE = 128
EP = 160768
EPW = EP // NW
GB_LEN = 352
RB = 1000


def _mesh():
    return plsc.VectorSubcoreMesh(core_axis_name="c", subcore_axis_name="s")


@functools.partial(
    pl.kernel,
    out_type=[
        jax.ShapeDtypeStruct((NP, F), jnp.float32),
        jax.ShapeDtypeStruct((NP, F), jnp.float32),
        jax.ShapeDtypeStruct((NP, F), jnp.float32),
        jax.ShapeDtypeStruct((NP, F), jnp.float32),
        jax.ShapeDtypeStruct((NP, 16), jnp.float32),
    ],
    mesh=_mesh(),
    scratch_types=[
        pltpu.VMEM((G, F), jnp.float32),
        pltpu.VMEM((G, F), jnp.float32),
        pltpu.VMEM((G, F), jnp.float32),
        pltpu.VMEM((G, F), jnp.float32),
        pltpu.VMEM((G, 16), jnp.float32),
        pltpu.VMEM((CE, F), jnp.float32),
        pltpu.VMEM((CE,), jnp.int32),
        pltpu.VMEM((CE + 16,), jnp.int32),
        pltpu.VMEM((GB_LEN,), jnp.int32),
        pltpu.SemaphoreType.DMA,
    ],
)
def _seg_kernel(b_hbm, srcs_hbm, dsts_hbm, gb_hbm,
                sb_hbm, sb2_hbm, mx_hbm, mn_hbm, cnt_hbm,
                st_sb, st_sb2, st_mx, st_mn, st_cnt,
                rows, sidx, didx, gb_v, sem):
    cid = lax.axis_index("c")
    sid = lax.axis_index("s")
    wid = sid * 2 + cid
    zeros16 = jnp.zeros((16,), jnp.float32)
    ones16 = jnp.ones((16,), jnp.float32)
    neg16 = jnp.full((16,), -jnp.inf, jnp.float32)
    pos16 = jnp.full((16,), jnp.inf, jnp.float32)
    pltpu.sync_copy(gb_hbm, gb_v)
    for g in range(GPW):
        gid = wid * GPW + g
        gbase = gid * G

        def zinit(i, _):
            st_cnt[i] = zeros16
            for fc in range(F // 16):
                sl = pl.ds(fc * 16, 16)
                st_sb[i, sl] = zeros16
                st_sb2[i, sl] = zeros16
                st_mx[i, sl] = neg16
                st_mn[i, sl] = pos16
            return 0

        lax.fori_loop(0, G, zinit, 0)
        gbv = gb_v[pl.ds(gid, 16)]
        lo = gbv[0]
        hi = gbv[1]
        start = lax.bitwise_and(lo, jnp.int32(-8))
        nch = lax.div(hi - start + jnp.int32(CE - 1), jnp.int32(CE))

        def chunk_body(ci, _):
            base = pl.multiple_of(start + ci * CE, 8)
            pltpu.sync_copy(srcs_hbm.at[pl.ds(base, CE)], sidx)
            pltpu.sync_copy(dsts_hbm.at[pl.ds(base, CE)],
                            didx.at[pl.ds(0, CE)])
            pltpu.async_copy(b_hbm.at[sidx], rows, sem).wait()

            def edge_body(e, __):
                r = didx[pl.ds(e, 16)][0]
                ok = jnp.logical_and(r >= gbase, r < gbase + G)

                @pl.when(ok)
                def _():
                    lr = r - gbase
                    plsc.addupdate(st_cnt.at[lr], ones16)
                    for fc in range(F // 16):
                        sl = pl.ds(fc * 16, 16)
                        v = rows[e, sl]
                        plsc.addupdate(st_sb.at[lr, sl], v)
                        plsc.addupdate(st_sb2.at[lr, sl], v * v)
                        st_mx[lr, sl] = jnp.maximum(st_mx[lr, sl], v)
                        st_mn[lr, sl] = jnp.minimum(st_mn[lr, sl], v)

                return 0

            lax.fori_loop(0, CE, edge_body, 0)
            return 0

        lax.fori_loop(0, nch, chunk_body, 0)
        pltpu.sync_copy(st_sb, sb_hbm.at[pl.ds(gbase, G)])
        pltpu.sync_copy(st_sb2, sb2_hbm.at[pl.ds(gbase, G)])
        pltpu.sync_copy(st_mx, mx_hbm.at[pl.ds(gbase, G)])
        pltpu.sync_copy(st_mn, mn_hbm.at[pl.ds(gbase, G)])
        pltpu.sync_copy(st_cnt, cnt_hbm.at[pl.ds(gbase, G)])


CS = NP // 16


@functools.partial(
    pl.kernel,
    out_type=jax.ShapeDtypeStruct((2, NP), jnp.float32),
    mesh=_mesh(),
    scratch_types=[
        pltpu.VMEM((NP + 16,), jnp.float32),
        pltpu.VMEM((EPW + 16,), jnp.int32),
        pltpu.VMEM((CS,), jnp.float32),
        pltpu.VMEM((CS,), jnp.float32),
        pltpu.VMEM_SHARED((16, NP), jnp.float32),
    ],
)
def _deg_kernel(srch_hbm, out_hbm, hist, srcs_v, tmp, acc, sh):
    cid = lax.axis_index("c")
    sid = lax.axis_index("s")
    wid = sid * 2 + cid
    zeros16 = jnp.zeros((16,), jnp.float32)
    lane = lax.iota(jnp.int32, 16)
    e0 = jnp.where(lane == 0, 1.0, 0.0).astype(jnp.float32)

    def z(i, _):
        hist[pl.ds(i * 16, 16)] = zeros16
        return 0

    lax.fori_loop(0, (NP + 16) // 16, z, 0)
    pltpu.sync_copy(srch_hbm.at[pl.ds(wid * EPW, EPW)],
                    srcs_v.at[pl.ds(0, EPW)])

    def edge(e, _):
        r = srcs_v[pl.ds(e, 16)][0]
        plsc.addupdate(hist.at[pl.ds(r, 16)], e0)
        return 0

    lax.fori_loop(0, EPW, edge, 0)
    pltpu.sync_copy(hist.at[pl.ds(0, NP)], sh.at[sid])
    plsc.subcore_barrier()
    base = sid * CS

    def zacc(i, _):
        acc[pl.ds(i * 16, 16)] = zeros16
        return 0

    lax.fori_loop(0, CS // 16, zacc, 0)
    for t in range(16):
        pltpu.sync_copy(sh.at[t, pl.ds(base, CS)], tmp)

        def addt(i, _):
            sl = pl.ds(i * 16, 16)
            plsc.addupdate(acc.at[sl], tmp[sl])
            return 0

        lax.fori_loop(0, CS // 16, addt, 0)
    pltpu.sync_copy(acc, out_hbm.at[cid, pl.ds(base, CS)])


def _ab_body(h_ref, w_ref, b_ref, a_ref, bo_ref):
    res = jnp.dot(h_ref[...], w_ref[...],
                  preferred_element_type=jnp.float32) + b_ref[...]
    a_ref[...] = res[:, :F]
    bo_ref[...] = res[:, F:]


_ab_call = pl.pallas_call(
    _ab_body,
    grid=(N // RB,),
    in_specs=[
        pl.BlockSpec((RB, F), lambda i: (i, 0)),
        pl.BlockSpec((F, 2 * F), lambda i: (0, 0)),
        pl.BlockSpec((1, 2 * F), lambda i: (0, 0)),
    ],
    out_specs=[
        pl.BlockSpec((RB, F), lambda i: (i, 0)),
        pl.BlockSpec((RB, F), lambda i: (i, 0)),
    ],
    out_shape=[
        jax.ShapeDtypeStruct((N, F), jnp.float32),
        jax.ShapeDtypeStruct((N, F), jnp.float32),
    ],
)


def _post_body(h_ref, a_ref, sb_ref, sb2_ref, mx_ref, mn_ref, cnt_ref,
               deg2_ref, q_ref, qb_ref, lw_ref, lb_ref, o_ref):
    h = h_ref[...]
    A = a_ref[...]
    cnt = cnt_ref[:, :1]
    cnt_c = jnp.maximum(cnt, 1.0)
    has = cnt > 0.0
    SB = sb_ref[...]
    SB2 = sb2_ref[...]
    mean = (cnt * A + SB) / cnt_c
    mean2 = (cnt * A * A + 2.0 * A * SB + SB2) / cnt_c
    var = mean2 - mean * mean
    std = jnp.sqrt(jnp.maximum(var, 0.0) + 1e-5)
    mx = jnp.where(has, A + mx_ref[...], 0.0)
    mn = jnp.where(has, A + mn_ref[...], 0.0)
    dg = deg2_ref[:, 0:1] + deg2_ref[:, 1:2]
    logd = jnp.log(jnp.maximum(dg, 1.0) + 1.0)
    amp = logd * (1.0 / ADL)
    att = ADL / logd
    q = q_ref[...]

    def dot(u, j):
        return jnp.dot(u, q[j], preferred_element_type=jnp.float32)

    base = (dot(h, 0) + dot(mean, 1) + dot(mx, 2) + dot(mn, 3) + dot(std, 4))
    ac = dot(mean, 5) + dot(mx, 6) + dot(mn, 7) + dot(std, 8)
    at2 = dot(mean, 9) + dot(mx, 10) + dot(mn, 11) + dot(std, 12)
    pre = base + amp * ac + att * at2 + qb_ref[...]
    out = jnp.dot(pre, lw_ref[...],
                  preferred_element_type=jnp.float32) + lb_ref[...] + h
    o_ref[...] = out


_post_call = pl.pallas_call(
    _post_body,
    grid=(N // RB,),
    in_specs=[
        pl.BlockSpec((RB, F), lambda i: (i, 0)),
        pl.BlockSpec((RB, F), lambda i: (i, 0)),
        pl.BlockSpec((RB, F), lambda i: (i, 0)),
        pl.BlockSpec((RB, F), lambda i: (i, 0)),
        pl.BlockSpec((RB, F), lambda i: (i, 0)),
        pl.BlockSpec((RB, F), lambda i: (i, 0)),
        pl.BlockSpec((RB, 16), lambda i: (i, 0)),
        pl.BlockSpec((RB, 2), lambda i: (i, 0)),
        pl.BlockSpec((13, F, F), lambda i: (0, 0, 0)),
        pl.BlockSpec((1, F), lambda i: (0, 0)),
        pl.BlockSpec((F, F), lambda i: (0, 0)),
        pl.BlockSpec((1, F), lambda i: (0, 0)),
    ],
    out_specs=pl.BlockSpec((RB, F), lambda i: (i, 0)),
    out_shape=jax.ShapeDtypeStruct((N, F), jnp.float32),
)


def kernel(x, edge_index, pre_W, pre_b, post_W, post_b, lin_W, lin_b):
    x = x.astype(jnp.float32)
    src = edge_index[0]
    dst = edge_index[1]
    dsts_s, srcs_s = lax.sort_key_val(dst, src)
    dsts_p = jnp.concatenate(
        [dsts_s, jnp.full((EP - E,), NP, jnp.int32)])
    srcs_p = jnp.concatenate(
        [srcs_s, jnp.zeros((EP - E,), jnp.int32)])
    grid_vals = jnp.arange(0, NP + 1, G, dtype=jnp.int32)
    gb = jnp.searchsorted(dsts_p, grid_vals).astype(jnp.int32)
    gb = jnp.concatenate(
        [gb, jnp.full((GB_LEN - grid_vals.shape[0],), E, jnp.int32)])
    srch = jnp.concatenate(
        [src, jnp.full((EP - E,), NP - 1, jnp.int32)])

    deg2 = jnp.transpose(_deg_kernel(srch))

    h = x
    for i in range(L):
        wab = jnp.concatenate([pre_W[i, :F, :], pre_W[i, F:, :]], axis=1)
        bab = jnp.concatenate(
            [jnp.zeros((F,), jnp.float32), pre_b[i]]).reshape(1, 2 * F)
        A, B = _ab_call(h, wab, bab)
        SB, SB2, MX, MN, CNT = _seg_kernel(B, srcs_p, dsts_p, gb)
        q = post_W[i].reshape(13, F, F)
        h = _post_call(h, A, SB, SB2, MX, MN, CNT, deg2, q,
                       post_b[i].reshape(1, F), lin_W[i],
                       lin_b[i].reshape(1, F))
    return h


# speedup vs baseline: 4.1626x; 1.2399x over previous
import functools

import jax
import jax.numpy as jnp
import numpy as np
from jax import lax
from jax.experimental import pallas as pl
from jax.experimental.pallas import tpu as pltpu
from jax.experimental.pallas import tpu_sc as plsc

N = 10000
E = 160000
F = 256
L = 3
AVG_DEG = 16.0

_hist = np.array([1.0] * 10 + [float(int(AVG_DEG) * 2)] * 10)
_bins = np.arange(_hist.shape[0])
ADL = float((np.log(_bins + 1.0) * _hist).sum() / _hist.sum())

NP = 10240
G = 32
NGRP = NP // G
NW = 32
GPW = NGRP // NW
CE = 128
EP = 160768
EPW = EP // NW
GB_LEN = 352
RB = 1000


def _mesh():
    return plsc.VectorSubcoreMesh(core_axis_name="c", subcore_axis_name="s")


FH = F // 2
NC8 = FH // 16


@functools.partial(
    pl.kernel,
    out_type=[
        jax.ShapeDtypeStruct((NP, F), jnp.float32),
        jax.ShapeDtypeStruct((NP, F), jnp.float32),
        jax.ShapeDtypeStruct((NP, F), jnp.float32),
        jax.ShapeDtypeStruct((NP, F), jnp.float32),
        jax.ShapeDtypeStruct((NP, 16), jnp.float32),
    ],
    mesh=_mesh(),
    scratch_types=[
        pltpu.VMEM((G, F), jnp.float32),
        pltpu.VMEM((G, F), jnp.float32),
        pltpu.VMEM((G, F), jnp.float32),
        pltpu.VMEM((G, F), jnp.float32),
        pltpu.VMEM((G, 16), jnp.float32),
        pltpu.VMEM((CE, FH), jnp.float32),
        pltpu.VMEM((CE,), jnp.int32),
        pltpu.VMEM((CE + 16,), jnp.int32),
        pltpu.VMEM((GB_LEN,), jnp.int32),
        pltpu.SemaphoreType.DMA,
    ],
)
def _seg_kernel(b0_hbm, b1_hbm, srcs_hbm, dsts_hbm, gb_hbm,
                sb_hbm, sb2_hbm, mx_hbm, mn_hbm, cnt_hbm,
                st_sb, st_sb2, st_mx, st_mn, st_cnt,
                rows, sidx, didx, gb_v, sem):
    cid = lax.axis_index("c")
    sid = lax.axis_index("s")
    wid = sid * 2 + cid
    zeros16 = jnp.zeros((16,), jnp.float32)
    ones16 = jnp.ones((16,), jnp.float32)
    neg16 = jnp.full((16,), -jnp.inf, jnp.float32)
    pos16 = jnp.full((16,), jnp.inf, jnp.float32)
    pltpu.sync_copy(gb_hbm, gb_v)
    zrow = jnp.zeros((1, 16), jnp.float32)
    zt = tuple(zrow for _ in range(NC8))
    def group_body(g, _g):
        gid = wid * GPW + g
        gbase = pl.multiple_of(gid * G, 8)

        def zinit(i, _):
            st_cnt[i] = zeros16
            for fc in range(F // 16):
                sl = pl.ds(fc * 16, 16)
                st_sb[i, sl] = zeros16
                st_sb2[i, sl] = zeros16
                st_mx[i, sl] = neg16
                st_mn[i, sl] = pos16
            return 0

        lax.fori_loop(0, G, zinit, 0)
        gbv = gb_v[pl.ds(gid, 16)]
        lo = gbv[0]
        hi = gbv[1]
        start = lax.bitwise_and(lo, jnp.int32(-8))
        nch = lax.div(hi - start + jnp.int32(CE - 1), jnp.int32(CE))

        for p in range(2):
            bp_hbm = b0_hbm if p == 0 else b1_hbm
            foff = p * FH

            def flush(pred, cur_r, ccnt, sb, sb2, mx, mn):
                @pl.when(pred)
                def _():
                    lr = pl.ds(cur_r - gbase, 1)
                    if p == 0:
                        st_cnt[lr] = ccnt
                    for c in range(NC8):
                        sl = pl.ds(foff + c * 16, 16)
                        st_sb[lr, sl] = sb[c]
                        st_sb2[lr, sl] = sb2[c]
                        st_mx[lr, sl] = mx[c]
                        st_mn[lr, sl] = mn[c]

            def chunk_body(ci, carry):
                base = pl.multiple_of(start + ci * CE, 8)
                pltpu.sync_copy(srcs_hbm.at[pl.ds(base, CE)], sidx)
                pltpu.sync_copy(dsts_hbm.at[pl.ds(base, CE)],
                                didx.at[pl.ds(0, CE)])
                pltpu.async_copy(bp_hbm.at[sidx], rows, sem).wait()

                def edge_body(e, ec):
                    cur_r, ccnt, sb, sb2, mx, mn = ec
                    r = didx[pl.ds(e, 16)][0]
                    vs = [rows[pl.ds(e, 1), pl.ds(c * 16, 16)]
                          for c in range(NC8)]
                    is_new = r != cur_r
                    in_r = jnp.logical_and(r >= gbase, r < gbase + G)
                    flush(jnp.logical_and(is_new, cur_r >= 0),
                          cur_r, ccnt, sb, sb2, mx, mn)
                    ncur = jnp.where(
                        is_new, jnp.where(in_r, r, jnp.int32(-1)), cur_r)
                    nccnt = jnp.where(is_new, zrow, ccnt) + 1.0
                    nsb = tuple(jnp.where(is_new, 0.0, a) + v
                                for a, v in zip(sb, vs))
                    nsb2 = tuple(jnp.where(is_new, 0.0, a) + v * v
                                 for a, v in zip(sb2, vs))
                    nmx = tuple(jnp.maximum(jnp.where(is_new, -jnp.inf, a), v)
                                for a, v in zip(mx, vs))
                    nmn = tuple(jnp.minimum(jnp.where(is_new, jnp.inf, a), v)
                                for a, v in zip(mn, vs))
                    return (ncur, nccnt, nsb, nsb2, nmx, nmn)

                return lax.fori_loop(0, CE, edge_body, carry)

            fcarry = lax.fori_loop(
                0, nch, chunk_body,
                (jnp.int32(-1), zrow, zt, zt, zt, zt))
            flush(fcarry[0] >= 0, *fcarry)

        pltpu.sync_copy(st_sb, sb_hbm.at[pl.ds(gbase, G)])
        pltpu.sync_copy(st_sb2, sb2_hbm.at[pl.ds(gbase, G)])
        pltpu.sync_copy(st_mx, mx_hbm.at[pl.ds(gbase, G)])
        pltpu.sync_copy(st_mn, mn_hbm.at[pl.ds(gbase, G)])
        pltpu.sync_copy(st_cnt, cnt_hbm.at[pl.ds(gbase, G)])
        return 0

    lax.fori_loop(0, GPW, group_body, 0)


CS = NP // 16


@functools.partial(
    pl.kernel,
    out_type=jax.ShapeDtypeStruct((2, NP), jnp.float32),
    mesh=_mesh(),
    scratch_types=[
        pltpu.VMEM((NP + 16,), jnp.float32),
        pltpu.VMEM((EPW + 16,), jnp.int32),
        pltpu.VMEM((CS,), jnp.float32),
        pltpu.VMEM((CS,), jnp.float32),
        pltpu.VMEM_SHARED((16, NP), jnp.float32),
    ],
)
def _deg_kernel(srch_hbm, out_hbm, hist, srcs_v, tmp, acc, sh):
    cid = lax.axis_index("c")
    sid = lax.axis_index("s")
    wid = sid * 2 + cid
    zeros16 = jnp.zeros((16,), jnp.float32)
    lane = lax.iota(jnp.int32, 16)
    e0 = jnp.where(lane == 0, 1.0, 0.0).astype(jnp.float32)

    def z(i, _):
        hist[pl.ds(i * 16, 16)] = zeros16
        return 0

    lax.fori_loop(0, (NP + 16) // 16, z, 0)
    pltpu.sync_copy(srch_hbm.at[pl.ds(wid * EPW, EPW)],
                    srcs_v.at[pl.ds(0, EPW)])

    def edge(e, _):
        r = srcs_v[pl.ds(e, 16)][0]
        plsc.addupdate(hist.at[pl.ds(r, 16)], e0)
        return 0

    lax.fori_loop(0, EPW, edge, 0)
    pltpu.sync_copy(hist.at[pl.ds(0, NP)], sh.at[sid])
    plsc.subcore_barrier()
    base = sid * CS

    def zacc(i, _):
        acc[pl.ds(i * 16, 16)] = zeros16
        return 0

    lax.fori_loop(0, CS // 16, zacc, 0)
    for t in range(16):
        pltpu.sync_copy(sh.at[t, pl.ds(base, CS)], tmp)

        def addt(i, _):
            sl = pl.ds(i * 16, 16)
            plsc.addupdate(acc.at[sl], tmp[sl])
            return 0

        lax.fori_loop(0, CS // 16, addt, 0)
    pltpu.sync_copy(acc, out_hbm.at[cid, pl.ds(base, CS)])


def _ab_body(h_ref, w_ref, b_ref, a_ref, b0_ref, b1_ref):
    res = jnp.dot(h_ref[...], w_ref[...],
                  preferred_element_type=jnp.float32) + b_ref[...]
    a_ref[...] = res[:, :F]
    b0_ref[...] = res[:, F:F + FH]
    b1_ref[...] = res[:, F + FH:]


_ab_call = pl.pallas_call(
    _ab_body,
    grid=(N // RB,),
    in_specs=[
        pl.BlockSpec((RB, F), lambda i: (i, 0)),
        pl.BlockSpec((F, 2 * F), lambda i: (0, 0)),
        pl.BlockSpec((1, 2 * F), lambda i: (0, 0)),
    ],
    out_specs=[
        pl.BlockSpec((RB, F), lambda i: (i, 0)),
        pl.BlockSpec((RB, FH), lambda i: (i, 0)),
        pl.BlockSpec((RB, FH), lambda i: (i, 0)),
    ],
    out_shape=[
        jax.ShapeDtypeStruct((N, F), jnp.float32),
        jax.ShapeDtypeStruct((N, FH), jnp.float32),
        jax.ShapeDtypeStruct((N, FH), jnp.float32),
    ],
)


def _post_body(h_ref, a_ref, sb_ref, sb2_ref, mx_ref, mn_ref, cnt_ref,
               deg2_ref, q_ref, qb_ref, lw_ref, lb_ref, o_ref):
    h = h_ref[...]
    A = a_ref[...]
    cnt = cnt_ref[:, :1]
    cnt_c = jnp.maximum(cnt, 1.0)
    has = cnt > 0.0
    SB = sb_ref[...]
    SB2 = sb2_ref[...]
    mean = (cnt * A + SB) / cnt_c
    mean2 = (cnt * A * A + 2.0 * A * SB + SB2) / cnt_c
    var = mean2 - mean * mean
    std = jnp.sqrt(jnp.maximum(var, 0.0) + 1e-5)
    mx = jnp.where(has, A + mx_ref[...], 0.0)
    mn = jnp.where(has, A + mn_ref[...], 0.0)
    dg = deg2_ref[:, 0:1] + deg2_ref[:, 1:2]
    logd = jnp.log(jnp.maximum(dg, 1.0) + 1.0)
    amp = logd * (1.0 / ADL)
    att = ADL / logd
    q = q_ref[...]

    def dot(u, j):
        return jnp.dot(u, q[j], preferred_element_type=jnp.float32)

    base = (dot(h, 0) + dot(mean, 1) + dot(mx, 2) + dot(mn, 3) + dot(std, 4))
    ac = dot(mean, 5) + dot(mx, 6) + dot(mn, 7) + dot(std, 8)
    at2 = dot(mean, 9) + dot(mx, 10) + dot(mn, 11) + dot(std, 12)
    pre = base + amp * ac + att * at2 + qb_ref[...]
    out = jnp.dot(pre, lw_ref[...],
                  preferred_element_type=jnp.float32) + lb_ref[...] + h
    o_ref[...] = out


_post_call = pl.pallas_call(
    _post_body,
    grid=(N // RB,),
    in_specs=[
        pl.BlockSpec((RB, F), lambda i: (i, 0)),
        pl.BlockSpec((RB, F), lambda i: (i, 0)),
        pl.BlockSpec((RB, F), lambda i: (i, 0)),
        pl.BlockSpec((RB, F), lambda i: (i, 0)),
        pl.BlockSpec((RB, F), lambda i: (i, 0)),
        pl.BlockSpec((RB, F), lambda i: (i, 0)),
        pl.BlockSpec((RB, 16), lambda i: (i, 0)),
        pl.BlockSpec((RB, 2), lambda i: (i, 0)),
        pl.BlockSpec((13, F, F), lambda i: (0, 0, 0)),
        pl.BlockSpec((1, F), lambda i: (0, 0)),
        pl.BlockSpec((F, F), lambda i: (0, 0)),
        pl.BlockSpec((1, F), lambda i: (0, 0)),
    ],
    out_specs=pl.BlockSpec((RB, F), lambda i: (i, 0)),
    out_shape=jax.ShapeDtypeStruct((N, F), jnp.float32),
)


def kernel(x, edge_index, pre_W, pre_b, post_W, post_b, lin_W, lin_b):
    x = x.astype(jnp.float32)
    src = edge_index[0]
    dst = edge_index[1]
    dsts_s, srcs_s = lax.sort_key_val(dst, src)
    dsts_p = jnp.concatenate(
        [dsts_s, jnp.full((EP - E,), NP, jnp.int32)])
    srcs_p = jnp.concatenate(
        [srcs_s, jnp.zeros((EP - E,), jnp.int32)])
    grid_vals = jnp.arange(0, NP + 1, G, dtype=jnp.int32)
    gb = jnp.searchsorted(dsts_p, grid_vals).astype(jnp.int32)
    gb = jnp.concatenate(
        [gb, jnp.full((GB_LEN - grid_vals.shape[0],), E, jnp.int32)])
    srch = jnp.concatenate(
        [src, jnp.full((EP - E,), NP - 1, jnp.int32)])

    deg2 = jnp.transpose(_deg_kernel(srch))

    h = x
    for i in range(L):
        wab = jnp.concatenate([pre_W[i, :F, :], pre_W[i, F:, :]], axis=1)
        bab = jnp.concatenate(
            [jnp.zeros((F,), jnp.float32), pre_b[i]]).reshape(1, 2 * F)
        A, B0, B1 = _ab_call(h, wab, bab)
        SB, SB2, MX, MN, CNT = _seg_kernel(B0, B1, srcs_p, dsts_p, gb)
        q = post_W[i].reshape(13, F, F)
        h = _post_call(h, A, SB, SB2, MX, MN, CNT, deg2, q,
                       post_b[i].reshape(1, F), lin_W[i],
                       lin_b[i].reshape(1, F))
    return h
